# baseline (speedup 1.0000x reference)
import jax
import jax.numpy as jnp
from jax import lax
from jax.experimental import pallas as pl

TOPK = 8
NG = 8
GSZ = 8
KG = 4
NE = 64
DIN = 1024


def _top2_merge(m1a, m2a, m1b, m2b):
    return (jnp.maximum(m1a, m1b),
            jnp.maximum(jnp.minimum(m1a, m1b), jnp.maximum(m2a, m2b)))


def _gate_block(x_ref, w_ref, b_ref, bias_ref, wout_ref, iout_ref):
    xb = x_ref[...]
    W = w_ref[...]
    s_lin = lax.dot_general(W, xb, (((1,), (1,)), ((), ())),
                            preferred_element_type=jnp.float32)
    s_lin = s_lin + b_ref[...]
    s2w = jax.nn.sigmoid(s_lin)
    score = s2w + bias_ref[...]
    bB = score.shape[1]
    neg = jnp.float32(-jnp.inf)

    gs_rows = []
    for g in range(NG):
        v = score[g * GSZ:(g + 1) * GSZ]
        m1, m2 = _top2_merge(v[0:4], jnp.full_like(v[0:4], neg),
                             v[4:8], jnp.full_like(v[0:4], neg))
        m1, m2 = _top2_merge(m1[0:2], m2[0:2], m1[2:4], m2[2:4])
        m1, m2 = _top2_merge(m1[0:1], m2[0:1], m1[1:2], m2[1:2])
        gs_rows.append(m1 + m2)
    gs = jnp.concatenate(gs_rows, axis=0)

    rowg = lax.broadcasted_iota(jnp.int32, (NG, bB), 0)
    rank = jnp.zeros((NG, bB), jnp.int32)
    for h in range(NG):
        gh = gs[h:h + 1]
        rank = rank + (gh > gs).astype(jnp.int32)
        rank = rank + ((gh == gs) & (h < rowg)).astype(jnp.int32)
    keep = (rank < KG).astype(jnp.float32)
    mask = jnp.concatenate(
        [jnp.broadcast_to(keep[g:g + 1], (GSZ, bB)) for g in range(NG)],
        axis=0)
    score_f = score * mask
    s2w_f = s2w * mask

    row = lax.broadcasted_iota(jnp.int32, (NE, bB), 0)
    cur = score_f
    wrows, irows = [], []
    for _ in range(TOPK):
        m = jnp.max(cur, axis=0, keepdims=True)
        lsel = jnp.min(jnp.where(cur == m, row, NE), axis=0, keepdims=True)
        hit = row == lsel
        wrows.append(jnp.max(jnp.where(hit, s2w_f, neg), axis=0, keepdims=True))
        irows.append(lsel)
        cur = jnp.where(hit, neg, cur)
    wout_ref[...] = jnp.concatenate(wrows, axis=0).T
    iout_ref[...] = jnp.concatenate(irows, axis=0).T


def kernel(x, W, b, bias):
    B = x.shape[0]
    bB = 4096
    b2 = b.reshape(NE, 1)
    bias2 = bias.reshape(NE, 1)
    wout, iout = pl.pallas_call(
        _gate_block,
        grid=(B // bB,),
        in_specs=[
            pl.BlockSpec((bB, DIN), lambda i: (i, 0)),
            pl.BlockSpec((NE, DIN), lambda i: (0, 0)),
            pl.BlockSpec((NE, 1), lambda i: (0, 0)),
            pl.BlockSpec((NE, 1), lambda i: (0, 0)),
        ],
        out_specs=[
            pl.BlockSpec((bB, TOPK), lambda i: (i, 0)),
            pl.BlockSpec((bB, TOPK), lambda i: (i, 0)),
        ],
        out_shape=[
            jax.ShapeDtypeStruct((B, TOPK), jnp.float32),
            jax.ShapeDtypeStruct((B, TOPK), jnp.int32),
        ],
    )(x, W, b2, bias2)
    return wout, iout
